# Initial kernel scaffold; baseline (speedup 1.0000x reference)
import jax
import jax.numpy as jnp
from jax.experimental import pallas as pl


def kernel(x, Wr, W1, b1, W2, b2):
    raise NotImplementedError("write your pallas kernel here")


# speedup vs baseline: 3.1395x; 3.1395x over previous
import functools

import jax
import jax.numpy as jnp
from jax import lax
from jax.experimental import pallas as pl
from jax.experimental.pallas import tpu as pltpu
from jax.experimental.pallas import tpu_sc as plsc

_B, _S, _D, _E, _K = 4, 2048, 1024, 8, 2
_T = _B * _S
_H = 4 * _D
_N = _T * _K
_LANES = 128

_BM = 512
_NB = _N // _BM + _E
_NPAD = _NB * _BM
_HT = 512

_CH = 1024

_NW = 32
_TW = _T // _NW
_CD = 64
_NCH = _TW // _CD


def _erf(x):
    a1, a2, a3, a4, a5 = (0.254829592, -0.284496736, 1.421413741,
                          -1.453152027, 1.061405429)
    p = 0.3275911
    s = jnp.sign(x)
    ax = jnp.abs(x)
    t = 1.0 / (1.0 + p * ax)
    poly = t * (a1 + t * (a2 + t * (a3 + t * (a4 + t * a5))))
    y = 1.0 - poly * jnp.exp(-ax * ax)
    return s * y


def _gelu(x):
    return 0.5 * x * (1.0 + _erf(x * 0.7071067811865476))


def _router_body(x_ref, wr_ref, e0_ref, e1_ref, wa_ref, wb_ref, r0_ref,
                 r1_ref, cnt_ref, carry):
    c = pl.program_id(0)

    @pl.when(c == 0)
    def _():
        carry[...] = jnp.zeros_like(carry)

    x = x_ref[...]
    logits = lax.dot_general(x, wr_ref[...], (((1,), (1,)), ((), ())),
                             preferred_element_type=jnp.float32)
    lane = lax.broadcasted_iota(jnp.int32, (_CH, _LANES), 1)
    neg = jnp.float32(-1e30)
    logits = jnp.where(lane < _E, logits, neg)

    m1 = jnp.max(logits, axis=1, keepdims=True)
    i1 = jnp.min(jnp.where(logits == m1, lane, _LANES), axis=1, keepdims=True)
    logits2 = jnp.where(lane == i1, neg, logits)
    m2 = jnp.max(logits2, axis=1, keepdims=True)
    i2 = jnp.min(jnp.where(logits2 == m2, lane, _LANES), axis=1, keepdims=True)

    s = jnp.exp(m2 - m1)
    w0 = 1.0 / (1.0 + s)
    w1 = 1.0 - w0

    ohA = (lane == i1)
    ohB = (lane == i2)
    ohAb = ohA.astype(jnp.bfloat16)
    ohBb = ohB.astype(jnp.bfloat16)
    row = lax.broadcasted_iota(jnp.int32, (_CH, _CH), 0)
    col = lax.broadcasted_iota(jnp.int32, (_CH, _CH), 1)
    lex = (col < row).astype(jnp.bfloat16)
    s0ex = lax.dot_general(lex, ohAb, (((1,), (0,)), ((), ())),
                           preferred_element_type=jnp.float32)
    s1ex = lax.dot_general(lex, ohBb, (((1,), (0,)), ((), ())),
                           preferred_element_type=jnp.float32)
    s0in = s0ex + ohA.astype(jnp.float32)

    carryv = carry[...]
    base0 = jnp.sum(jnp.where(ohA, carryv + s0ex + s1ex, 0.0), axis=1,
                    keepdims=True)
    base1 = jnp.sum(jnp.where(ohB, carryv + s0in + s1ex, 0.0), axis=1,
                    keepdims=True)

    e0_ref[...] = i1
    e1_ref[...] = i2
    wa_ref[...] = w0
    wb_ref[...] = w1
    r0_ref[...] = base0.astype(jnp.int32)
    r1_ref[...] = base1.astype(jnp.int32)

    newc = carryv + jnp.sum(ohA.astype(jnp.float32) + ohB.astype(jnp.float32),
                            axis=0, keepdims=True)
    carry[...] = newc
    cnt_ref[...] = newc


def _router_call(x_flat, wr_pad):
    nchunks = _T // _CH
    out_shapes = (
        jax.ShapeDtypeStruct((_T, 1), jnp.int32),
        jax.ShapeDtypeStruct((_T, 1), jnp.int32),
        jax.ShapeDtypeStruct((_T, 1), jnp.float32),
        jax.ShapeDtypeStruct((_T, 1), jnp.float32),
        jax.ShapeDtypeStruct((_T, 1), jnp.int32),
        jax.ShapeDtypeStruct((_T, 1), jnp.int32),
        jax.ShapeDtypeStruct((1, _LANES), jnp.float32),
    )
    col_spec = pl.BlockSpec((_CH, 1), lambda c: (c, 0))
    return pl.pallas_call(
        _router_body,
        grid=(nchunks,),
        in_specs=[
            pl.BlockSpec((_CH, _D), lambda c: (c, 0)),
            pl.BlockSpec((_LANES, _D), lambda c: (0, 0)),
        ],
        out_specs=(col_spec, col_spec, col_spec, col_spec, col_spec, col_spec,
                   pl.BlockSpec((1, _LANES), lambda c: (0, 0))),
        out_shape=out_shapes,
        scratch_shapes=[pltpu.VMEM((1, _LANES), jnp.float32)],
    )(x_flat, wr_pad)


def _offsets_body(cnt_ref, e0_ref, e1_ref, r0_ref, r1_ref, pos0_ref, pos1_ref,
                  bexp_ref):
    lane1 = lax.broadcasted_iota(jnp.int32, (1, _LANES), 1)
    cnt = cnt_ref[...].astype(jnp.int32)
    cnt = jnp.where(lane1 < _E, cnt, 0)
    nblk = (cnt + (_BM - 1)) // _BM
    tri = (lax.broadcasted_iota(jnp.int32, (_LANES, _LANES), 0)
           <= lax.broadcasted_iota(jnp.int32, (_LANES, _LANES), 1))
    endb = lax.dot_general(nblk.astype(jnp.float32), tri.astype(jnp.float32),
                           (((1,), (0,)), ((), ())),
                           preferred_element_type=jnp.float32)
    startb = endb - nblk.astype(jnp.float32)
    pstart = startb * jnp.float32(_BM)

    for e_ref, r_ref, p_ref in ((e0_ref, r0_ref, pos0_ref),
                                (e1_ref, r1_ref, pos1_ref)):
        e = e_ref[...]
        laneT = lax.broadcasted_iota(jnp.int32, (_T, _LANES), 1)
        base = jnp.sum(jnp.where(laneT == e, pstart, 0.0), axis=1,
                       keepdims=True)
        p_ref[...] = r_ref[...] + base.astype(jnp.int32)

    brow = lax.broadcasted_iota(jnp.int32, (_NB, _LANES), 0)
    ge = (brow >= endb.astype(jnp.int32)) & (lane1 < _E)
    be = jnp.sum(ge.astype(jnp.int32), axis=1, keepdims=True)
    bexp_ref[...] = jnp.minimum(be, _E - 1)


def _offsets_call(cnt, e0, e1, r0, r1):
    out_shapes = (
        jax.ShapeDtypeStruct((_T, 1), jnp.int32),
        jax.ShapeDtypeStruct((_T, 1), jnp.int32),
        jax.ShapeDtypeStruct((_NB, 1), jnp.int32),
    )
    return pl.pallas_call(
        _offsets_body,
        out_shape=out_shapes,
    )(cnt, e0, e1, r0, r1)


def _dispatch_body(x_hbm, pos0_hbm, pos1_hbm, xg_hbm, idx0_v, idx1_v, rows_v,
                   sem):
    wid = lax.axis_index("s") * 2 + lax.axis_index("c")
    base = wid * _TW
    pltpu.sync_copy(pos0_hbm.at[wid], idx0_v)
    pltpu.sync_copy(pos1_hbm.at[wid], idx1_v)
    for j in range(_NCH):
        pltpu.sync_copy(x_hbm.at[pl.ds(base + j * _CD, _CD)], rows_v)
        cp0 = pltpu.make_async_copy(rows_v, xg_hbm.at[idx0_v.at[j]], sem)
        cp1 = pltpu.make_async_copy(rows_v, xg_hbm.at[idx1_v.at[j]], sem)
        cp0.start()
        cp1.start()
        cp0.wait()
        cp1.wait()


def _dispatch_call(x_flat, pos0w, pos1w):
    mesh = plsc.VectorSubcoreMesh(core_axis_name="c", subcore_axis_name="s")
    f = functools.partial(
        pl.kernel,
        out_type=jax.ShapeDtypeStruct((_NPAD, _D), jnp.float32),
        mesh=mesh,
        scratch_types=[
            pltpu.VMEM((_NCH, _CD), jnp.int32),
            pltpu.VMEM((_NCH, _CD), jnp.int32),
            pltpu.VMEM((_CD, _D), jnp.float32),
            pltpu.SemaphoreType.DMA,
        ],
    )(_dispatch_body)
    return f(x_flat, pos0w, pos1w)


def _ffn_body(bmap_ref, xg_ref, w1_ref, b1_ref, w2_ref, b2_ref, y_ref):
    del bmap_ref
    ht = pl.program_id(1)
    x = xg_ref[...].astype(jnp.bfloat16)
    hpre = lax.dot_general(x, w1_ref[0], (((1,), (1,)), ((), ())),
                           preferred_element_type=jnp.float32) + b1_ref[0]
    hact = _gelu(hpre).astype(jnp.bfloat16)
    part = lax.dot_general(hact, w2_ref[0], (((1,), (1,)), ((), ())),
                           preferred_element_type=jnp.float32)

    @pl.when(ht == 0)
    def _():
        y_ref[...] = part + b2_ref[0]

    @pl.when(ht != 0)
    def _():
        y_ref[...] += part


def _ffn_call(bmap, xg, w1b, b1, w2b, b2):
    nht = _H // _HT
    b1r = b1.reshape(_E * nht, 1, _HT)
    b2r = b2.reshape(_E, 1, _D)
    grid_spec = pltpu.PrefetchScalarGridSpec(
        num_scalar_prefetch=1,
        grid=(_NB, nht),
        in_specs=[
            pl.BlockSpec((_BM, _D), lambda b, h, m: (b, 0)),
            pl.BlockSpec((1, _HT, _D), lambda b, h, m: (m[b], h, 0)),
            pl.BlockSpec((1, 1, _HT), lambda b, h, m: (m[b] * nht + h, 0, 0)),
            pl.BlockSpec((1, _D, _HT), lambda b, h, m: (m[b], 0, h)),
            pl.BlockSpec((1, 1, _D), lambda b, h, m: (m[b], 0, 0)),
        ],
        out_specs=pl.BlockSpec((_BM, _D), lambda b, h, m: (b, 0)),
    )
    return pl.pallas_call(
        _ffn_body,
        grid_spec=grid_spec,
        out_shape=jax.ShapeDtypeStruct((_NPAD, _D), jnp.float32),
    )(bmap, xg, w1b, b1r, w2b, b2r)


def _combine_body(y_hbm, pos0_hbm, pos1_hbm, y0_hbm, y1_hbm, idx0_v, idx1_v,
                  buf_v, sem):
    wid = lax.axis_index("s") * 2 + lax.axis_index("c")
    base = wid * _TW
    pltpu.sync_copy(pos0_hbm.at[wid], idx0_v)
    pltpu.sync_copy(pos1_hbm.at[wid], idx1_v)
    for j in range(_NCH):
        pltpu.make_async_copy(y_hbm.at[idx0_v.at[j]], buf_v, sem).start()
        pltpu.make_async_copy(y_hbm.at[idx0_v.at[j]], buf_v, sem).wait()
        pltpu.sync_copy(buf_v, y0_hbm.at[pl.ds(base + j * _CD, _CD)])
        pltpu.make_async_copy(y_hbm.at[idx1_v.at[j]], buf_v, sem).start()
        pltpu.make_async_copy(y_hbm.at[idx1_v.at[j]], buf_v, sem).wait()
        pltpu.sync_copy(buf_v, y1_hbm.at[pl.ds(base + j * _CD, _CD)])


def _combine_call(y, pos0w, pos1w):
    mesh = plsc.VectorSubcoreMesh(core_axis_name="c", subcore_axis_name="s")
    f = functools.partial(
        pl.kernel,
        out_type=(jax.ShapeDtypeStruct((_T, _D), jnp.float32),
                  jax.ShapeDtypeStruct((_T, _D), jnp.float32)),
        mesh=mesh,
        scratch_types=[
            pltpu.VMEM((_NCH, _CD), jnp.int32),
            pltpu.VMEM((_NCH, _CD), jnp.int32),
            pltpu.VMEM((_CD, _D), jnp.float32),
            pltpu.SemaphoreType.DMA,
        ],
    )(_combine_body)
    return f(y, pos0w, pos1w)


def _wsum_body(y0_ref, y1_ref, w0_ref, w1_ref, out_ref):
    out_ref[...] = y0_ref[...] * w0_ref[...] + y1_ref[...] * w1_ref[...]


def _wsum_call(y0, y1, w0, w1):
    nchunks = _T // _CH
    row_spec = pl.BlockSpec((_CH, _D), lambda c: (c, 0))
    col_spec = pl.BlockSpec((_CH, 1), lambda c: (c, 0))
    return pl.pallas_call(
        _wsum_body,
        grid=(nchunks,),
        in_specs=[row_spec, row_spec, col_spec, col_spec],
        out_specs=row_spec,
        out_shape=jax.ShapeDtypeStruct((_T, _D), jnp.float32),
    )(y0, y1, w0, w1)


def kernel(x, Wr, W1, b1, W2, b2):
    x_flat = x.reshape(_T, _D)
    wr_pad = jnp.zeros((_LANES, _D), jnp.float32).at[:_E].set(Wr)

    e0, e1, w0, w1, r0, r1, cnt = _router_call(x_flat, wr_pad)
    pos0, pos1, bexp = _offsets_call(cnt, e0, e1, r0, r1)

    bmap = bexp.reshape(_NB)
    pos0w = pos0.reshape(_NW, _NCH, _CD)
    pos1w = pos1.reshape(_NW, _NCH, _CD)

    xg = _dispatch_call(x_flat, pos0w, pos1w)

    w1b = W1.astype(jnp.bfloat16)
    w2b = W2.astype(jnp.bfloat16)
    y = _ffn_call(bmap, xg, w1b, b1, w2b, b2)

    y0, y1 = _combine_call(y, pos0w, pos1w)
    out = _wsum_call(y0, y1, w0, w1)
    return out.reshape(_B, _S, _D)


# speedup vs baseline: 3.9214x; 1.2490x over previous
import functools

import jax
import jax.numpy as jnp
from jax import lax
from jax.experimental import pallas as pl
from jax.experimental.pallas import tpu as pltpu
from jax.experimental.pallas import tpu_sc as plsc

_B, _S, _D, _E, _K = 4, 2048, 1024, 8, 2
_T = _B * _S
_H = 4 * _D
_N = _T * _K
_LANES = 128

_BM = 512
_NB = _N // _BM + _E
_NBE = _NB + 8
_NPAD = _NB * _BM
_HT = 512

_CH = 1024

_NW = 32
_TW = _T // _NW
_CD = 64
_NCH = _TW // _CD


def _erf(x):
    a1, a2, a3, a4, a5 = (0.254829592, -0.284496736, 1.421413741,
                          -1.453152027, 1.061405429)
    p = 0.3275911
    s = jnp.sign(x)
    ax = jnp.abs(x)
    t = 1.0 / (1.0 + p * ax)
    poly = t * (a1 + t * (a2 + t * (a3 + t * (a4 + t * a5))))
    y = 1.0 - poly * jnp.exp(-ax * ax)
    return s * y


def _gelu(x):
    return 0.5 * x * (1.0 + _erf(x * 0.7071067811865476))


def _router_body(x_ref, wr_ref, e0_ref, e1_ref, wa_ref, wb_ref, r0_ref,
                 r1_ref, cnt_ref, carry):
    c = pl.program_id(0)

    @pl.when(c == 0)
    def _():
        carry[...] = jnp.zeros_like(carry)

    x = x_ref[...]
    logits = lax.dot_general(x, wr_ref[...], (((1,), (1,)), ((), ())),
                             preferred_element_type=jnp.float32)
    lane = lax.broadcasted_iota(jnp.int32, (_CH, _LANES), 1)
    neg = jnp.float32(-1e30)
    logits = jnp.where(lane < _E, logits, neg)

    m1 = jnp.max(logits, axis=1, keepdims=True)
    i1 = jnp.min(jnp.where(logits == m1, lane, _LANES), axis=1, keepdims=True)
    logits2 = jnp.where(lane == i1, neg, logits)
    m2 = jnp.max(logits2, axis=1, keepdims=True)
    i2 = jnp.min(jnp.where(logits2 == m2, lane, _LANES), axis=1, keepdims=True)

    s = jnp.exp(m2 - m1)
    w0 = 1.0 / (1.0 + s)
    w1 = 1.0 - w0

    ohA = (lane == i1)
    ohB = (lane == i2)
    ohAb = ohA.astype(jnp.bfloat16)
    ohBb = ohB.astype(jnp.bfloat16)
    row = lax.broadcasted_iota(jnp.int32, (_CH, _CH), 0)
    col = lax.broadcasted_iota(jnp.int32, (_CH, _CH), 1)
    lex = (col < row).astype(jnp.bfloat16)
    s0ex = lax.dot_general(lex, ohAb, (((1,), (0,)), ((), ())),
                           preferred_element_type=jnp.float32)
    s1ex = lax.dot_general(lex, ohBb, (((1,), (0,)), ((), ())),
                           preferred_element_type=jnp.float32)
    s0in = s0ex + ohA.astype(jnp.float32)

    carryv = carry[...]
    base0 = jnp.sum(jnp.where(ohA, carryv + s0ex + s1ex, 0.0), axis=1,
                    keepdims=True)
    base1 = jnp.sum(jnp.where(ohB, carryv + s0in + s1ex, 0.0), axis=1,
                    keepdims=True)

    e0_ref[...] = i1
    e1_ref[...] = i2
    wa_ref[...] = w0
    wb_ref[...] = w1
    r0_ref[...] = base0.astype(jnp.int32)
    r1_ref[...] = base1.astype(jnp.int32)

    newc = carryv + jnp.sum(ohA.astype(jnp.float32) + ohB.astype(jnp.float32),
                            axis=0, keepdims=True)
    carry[...] = newc
    cnt_ref[...] = newc


def _router_call(x_flat, wr_pad):
    nchunks = _T // _CH
    out_shapes = (
        jax.ShapeDtypeStruct((_T, 1), jnp.int32),
        jax.ShapeDtypeStruct((_T, 1), jnp.int32),
        jax.ShapeDtypeStruct((_T, 1), jnp.float32),
        jax.ShapeDtypeStruct((_T, 1), jnp.float32),
        jax.ShapeDtypeStruct((_T, 1), jnp.int32),
        jax.ShapeDtypeStruct((_T, 1), jnp.int32),
        jax.ShapeDtypeStruct((1, _LANES), jnp.float32),
    )
    col_spec = pl.BlockSpec((_CH, 1), lambda c: (c, 0))
    return pl.pallas_call(
        _router_body,
        grid=(nchunks,),
        in_specs=[
            pl.BlockSpec((_CH, _D), lambda c: (c, 0)),
            pl.BlockSpec((_LANES, _D), lambda c: (0, 0)),
        ],
        out_specs=(col_spec, col_spec, col_spec, col_spec, col_spec, col_spec,
                   pl.BlockSpec((1, _LANES), lambda c: (0, 0))),
        out_shape=out_shapes,
        scratch_shapes=[pltpu.VMEM((1, _LANES), jnp.float32)],
    )(x_flat, wr_pad)


def _offsets_body(cnt_ref, e0_ref, e1_ref, r0_ref, r1_ref, pos0_ref, pos1_ref,
                  bexp_ref):
    lane1 = lax.broadcasted_iota(jnp.int32, (1, _LANES), 1)
    cnt = cnt_ref[...].astype(jnp.int32)
    cnt = jnp.where(lane1 < _E, cnt, 0)
    nblk = (cnt + (_BM - 1)) // _BM
    tri = (lax.broadcasted_iota(jnp.int32, (_LANES, _LANES), 0)
           <= lax.broadcasted_iota(jnp.int32, (_LANES, _LANES), 1))
    endb = lax.dot_general(nblk.astype(jnp.float32), tri.astype(jnp.float32),
                           (((1,), (0,)), ((), ())),
                           preferred_element_type=jnp.float32)
    startb = endb - nblk.astype(jnp.float32)
    pstart = startb * jnp.float32(_BM)

    for e_ref, r_ref, p_ref in ((e0_ref, r0_ref, pos0_ref),
                                (e1_ref, r1_ref, pos1_ref)):
        e = e_ref[...]
        laneT = lax.broadcasted_iota(jnp.int32, (_T, _LANES), 1)
        base = jnp.sum(jnp.where(laneT == e, pstart, 0.0), axis=1,
                       keepdims=True)
        p_ref[...] = r_ref[...] + base.astype(jnp.int32)

    brow = lax.broadcasted_iota(jnp.int32, (_NBE, _LANES), 0)
    ge = (brow >= endb.astype(jnp.int32)) & (lane1 < _E)
    be = jnp.minimum(jnp.sum(ge.astype(jnp.int32), axis=1, keepdims=True),
                     _E - 1)
    used = jnp.sum(jnp.where(lane1 < _E, nblk, 0)).astype(jnp.int32)
    rowi = lax.broadcasted_iota(jnp.int32, (_NBE, 1), 0)
    bexp_ref[...] = jnp.where(rowi < _NB, be, used)


def _offsets_call(cnt, e0, e1, r0, r1):
    out_shapes = (
        jax.ShapeDtypeStruct((_T, 1), jnp.int32),
        jax.ShapeDtypeStruct((_T, 1), jnp.int32),
        jax.ShapeDtypeStruct((_NBE, 1), jnp.int32),
    )
    return pl.pallas_call(
        _offsets_body,
        out_shape=out_shapes,
    )(cnt, e0, e1, r0, r1)


def _dispatch_body(x_hbm, pos0_hbm, pos1_hbm, xg_hbm, idx0_v, idx1_v, rows_v,
                   sem):
    wid = lax.axis_index("s") * 2 + lax.axis_index("c")
    base = wid * _TW
    pltpu.sync_copy(pos0_hbm.at[wid], idx0_v)
    pltpu.sync_copy(pos1_hbm.at[wid], idx1_v)
    for j in range(_NCH):
        pltpu.sync_copy(x_hbm.at[pl.ds(base + j * _CD, _CD)], rows_v)
        cp0 = pltpu.make_async_copy(rows_v, xg_hbm.at[idx0_v.at[j]], sem)
        cp1 = pltpu.make_async_copy(rows_v, xg_hbm.at[idx1_v.at[j]], sem)
        cp0.start()
        cp1.start()
        cp0.wait()
        cp1.wait()


def _dispatch_call(x_flat, pos0w, pos1w):
    mesh = plsc.VectorSubcoreMesh(core_axis_name="c", subcore_axis_name="s")
    f = functools.partial(
        pl.kernel,
        out_type=jax.ShapeDtypeStruct((_NPAD, _D), jnp.float32),
        mesh=mesh,
        scratch_types=[
            pltpu.VMEM((_NCH, _CD), jnp.int32),
            pltpu.VMEM((_NCH, _CD), jnp.int32),
            pltpu.VMEM((_CD, _D), jnp.float32),
            pltpu.SemaphoreType.DMA,
        ],
    )(_dispatch_body)
    return f(x_flat, pos0w, pos1w)


def _ffn_body(bmap_ref, xg_ref, w1_ref, b1_ref, w2_ref, b2_ref, y_ref):
    b = pl.program_id(0)
    used = bmap_ref[_NB]

    @pl.when(b < used)
    def _():
        x = xg_ref[...].astype(jnp.bfloat16)
        acc = b2_ref[0]
        for ht in range(_H // _HT):
            w1c = w1_ref[0, ht * _HT:(ht + 1) * _HT, :]
            hpre = lax.dot_general(x, w1c, (((1,), (1,)), ((), ())),
                                   preferred_element_type=jnp.float32)
            hpre = hpre + b1_ref[0, :, ht * _HT:(ht + 1) * _HT]
            hact = _gelu(hpre).astype(jnp.bfloat16)
            w2c = w2_ref[0, :, ht * _HT:(ht + 1) * _HT]
            acc = acc + lax.dot_general(hact, w2c, (((1,), (1,)), ((), ())),
                                        preferred_element_type=jnp.float32)
        y_ref[...] = acc


def _ffn_call(bmap, xg, w1b, b1, w2b, b2):
    b1r = b1.reshape(_E, 1, _H)
    b2r = b2.reshape(_E, 1, _D)
    grid_spec = pltpu.PrefetchScalarGridSpec(
        num_scalar_prefetch=1,
        grid=(_NB,),
        in_specs=[
            pl.BlockSpec((_BM, _D), lambda b, m: (b, 0)),
            pl.BlockSpec((1, _H, _D), lambda b, m: (m[b], 0, 0)),
            pl.BlockSpec((1, 1, _H), lambda b, m: (m[b], 0, 0)),
            pl.BlockSpec((1, _D, _H), lambda b, m: (m[b], 0, 0)),
            pl.BlockSpec((1, 1, _D), lambda b, m: (m[b], 0, 0)),
        ],
        out_specs=pl.BlockSpec((_BM, _D), lambda b, m: (b, 0)),
    )
    return pl.pallas_call(
        _ffn_body,
        grid_spec=grid_spec,
        out_shape=jax.ShapeDtypeStruct((_NPAD, _D), jnp.float32),
    )(bmap, xg, w1b, b1r, w2b, b2r)


def _combine_body(y_hbm, pos0_hbm, pos1_hbm, y0_hbm, y1_hbm, idx0_v, idx1_v,
                  buf_v, sem):
    wid = lax.axis_index("s") * 2 + lax.axis_index("c")
    base = wid * _TW
    pltpu.sync_copy(pos0_hbm.at[wid], idx0_v)
    pltpu.sync_copy(pos1_hbm.at[wid], idx1_v)
    for j in range(_NCH):
        pltpu.make_async_copy(y_hbm.at[idx0_v.at[j]], buf_v, sem).start()
        pltpu.make_async_copy(y_hbm.at[idx0_v.at[j]], buf_v, sem).wait()
        pltpu.sync_copy(buf_v, y0_hbm.at[pl.ds(base + j * _CD, _CD)])
        pltpu.make_async_copy(y_hbm.at[idx1_v.at[j]], buf_v, sem).start()
        pltpu.make_async_copy(y_hbm.at[idx1_v.at[j]], buf_v, sem).wait()
        pltpu.sync_copy(buf_v, y1_hbm.at[pl.ds(base + j * _CD, _CD)])


def _combine_call(y, pos0w, pos1w):
    mesh = plsc.VectorSubcoreMesh(core_axis_name="c", subcore_axis_name="s")
    f = functools.partial(
        pl.kernel,
        out_type=(jax.ShapeDtypeStruct((_T, _D), jnp.float32),
                  jax.ShapeDtypeStruct((_T, _D), jnp.float32)),
        mesh=mesh,
        scratch_types=[
            pltpu.VMEM((_NCH, _CD), jnp.int32),
            pltpu.VMEM((_NCH, _CD), jnp.int32),
            pltpu.VMEM((_CD, _D), jnp.float32),
            pltpu.SemaphoreType.DMA,
        ],
    )(_combine_body)
    return f(y, pos0w, pos1w)


def _wsum_body(y0_ref, y1_ref, w0_ref, w1_ref, out_ref):
    out_ref[...] = y0_ref[...] * w0_ref[...] + y1_ref[...] * w1_ref[...]


def _wsum_call(y0, y1, w0, w1):
    nchunks = _T // _CH
    row_spec = pl.BlockSpec((_CH, _D), lambda c: (c, 0))
    col_spec = pl.BlockSpec((_CH, 1), lambda c: (c, 0))
    return pl.pallas_call(
        _wsum_body,
        grid=(nchunks,),
        in_specs=[row_spec, row_spec, col_spec, col_spec],
        out_specs=row_spec,
        out_shape=jax.ShapeDtypeStruct((_T, _D), jnp.float32),
    )(y0, y1, w0, w1)


def kernel(x, Wr, W1, b1, W2, b2):
    x_flat = x.reshape(_T, _D)
    wr_pad = jnp.zeros((_LANES, _D), jnp.float32).at[:_E].set(Wr)

    e0, e1, w0, w1, r0, r1, cnt = _router_call(x_flat, wr_pad)
    pos0, pos1, bexp = _offsets_call(cnt, e0, e1, r0, r1)

    bmap = bexp.reshape(_NBE)
    pos0w = pos0.reshape(_NW, _NCH, _CD)
    pos1w = pos1.reshape(_NW, _NCH, _CD)

    xg = _dispatch_call(x_flat, pos0w, pos1w)

    w1b = W1.astype(jnp.bfloat16)
    w2b = W2.astype(jnp.bfloat16)
    y = _ffn_call(bmap, xg, w1b, b1, w2b, b2)

    y0, y1 = _combine_call(y, pos0w, pos1w)
    out = _wsum_call(y0, y1, w0, w1)
    return out.reshape(_B, _S, _D)


# speedup vs baseline: 4.9572x; 1.2641x over previous
import functools

import jax
import jax.numpy as jnp
from jax import lax
from jax.experimental import pallas as pl
from jax.experimental.pallas import tpu as pltpu
from jax.experimental.pallas import tpu_sc as plsc

_B, _S, _D, _E, _K = 4, 2048, 1024, 8, 2
_T = _B * _S
_H = 4 * _D
_N = _T * _K
_LANES = 128

_BM = 512
_NB = _N // _BM + _E
_NBE = _NB + 8
_NPAD = _NB * _BM
_HT = 512

_CH = 1024

_NW = 32
_TW = _T // _NW
_CD = 64
_NCH = _TW // _CD


def _erf(x):
    a1, a2, a3, a4, a5 = (0.254829592, -0.284496736, 1.421413741,
                          -1.453152027, 1.061405429)
    p = 0.3275911
    s = jnp.sign(x)
    ax = jnp.abs(x)
    t = 1.0 / (1.0 + p * ax)
    poly = t * (a1 + t * (a2 + t * (a3 + t * (a4 + t * a5))))
    y = 1.0 - poly * jnp.exp(-ax * ax)
    return s * y


def _gelu(x):
    u = 0.7978845608028654 * (x + 0.044715 * x * x * x)
    return 0.5 * x * (1.0 + jnp.tanh(u))


def _router_body(x_ref, wr_ref, e0_ref, e1_ref, wa_ref, wb_ref, r0_ref,
                 r1_ref, cnt_ref, carry):
    c = pl.program_id(0)

    @pl.when(c == 0)
    def _():
        carry[...] = jnp.zeros_like(carry)

    x = x_ref[...]
    logits = lax.dot_general(x, wr_ref[...], (((1,), (1,)), ((), ())),
                             preferred_element_type=jnp.float32)
    lane = lax.broadcasted_iota(jnp.int32, (_CH, _LANES), 1)
    neg = jnp.float32(-1e30)
    logits = jnp.where(lane < _E, logits, neg)

    m1 = jnp.max(logits, axis=1, keepdims=True)
    i1 = jnp.min(jnp.where(logits == m1, lane, _LANES), axis=1, keepdims=True)
    logits2 = jnp.where(lane == i1, neg, logits)
    m2 = jnp.max(logits2, axis=1, keepdims=True)
    i2 = jnp.min(jnp.where(logits2 == m2, lane, _LANES), axis=1, keepdims=True)

    s = jnp.exp(m2 - m1)
    w0 = 1.0 / (1.0 + s)
    w1 = 1.0 - w0

    ohA = (lane == i1)
    ohB = (lane == i2)
    ohAb = ohA.astype(jnp.bfloat16)
    ohBb = ohB.astype(jnp.bfloat16)
    row = lax.broadcasted_iota(jnp.int32, (_CH, _CH), 0)
    col = lax.broadcasted_iota(jnp.int32, (_CH, _CH), 1)
    lex = (col < row).astype(jnp.bfloat16)
    s0ex = lax.dot_general(lex, ohAb, (((1,), (0,)), ((), ())),
                           preferred_element_type=jnp.float32)
    s1ex = lax.dot_general(lex, ohBb, (((1,), (0,)), ((), ())),
                           preferred_element_type=jnp.float32)
    s0in = s0ex + ohA.astype(jnp.float32)

    carryv = carry[...]
    base0 = jnp.sum(jnp.where(ohA, carryv + s0ex + s1ex, 0.0), axis=1,
                    keepdims=True)
    base1 = jnp.sum(jnp.where(ohB, carryv + s0in + s1ex, 0.0), axis=1,
                    keepdims=True)

    e0_ref[...] = i1
    e1_ref[...] = i2
    wa_ref[...] = w0
    wb_ref[...] = w1
    r0_ref[...] = base0.astype(jnp.int32)
    r1_ref[...] = base1.astype(jnp.int32)

    newc = carryv + jnp.sum(ohA.astype(jnp.float32) + ohB.astype(jnp.float32),
                            axis=0, keepdims=True)
    carry[...] = newc
    cnt_ref[...] = newc


def _router_call(x_flat, wr_pad):
    nchunks = _T // _CH
    out_shapes = (
        jax.ShapeDtypeStruct((_T, 1), jnp.int32),
        jax.ShapeDtypeStruct((_T, 1), jnp.int32),
        jax.ShapeDtypeStruct((_T, 1), jnp.float32),
        jax.ShapeDtypeStruct((_T, 1), jnp.float32),
        jax.ShapeDtypeStruct((_T, 1), jnp.int32),
        jax.ShapeDtypeStruct((_T, 1), jnp.int32),
        jax.ShapeDtypeStruct((1, _LANES), jnp.float32),
    )
    col_spec = pl.BlockSpec((_CH, 1), lambda c: (c, 0))
    return pl.pallas_call(
        _router_body,
        grid=(nchunks,),
        in_specs=[
            pl.BlockSpec((_CH, _D), lambda c: (c, 0)),
            pl.BlockSpec((_LANES, _D), lambda c: (0, 0)),
        ],
        out_specs=(col_spec, col_spec, col_spec, col_spec, col_spec, col_spec,
                   pl.BlockSpec((1, _LANES), lambda c: (0, 0))),
        out_shape=out_shapes,
        scratch_shapes=[pltpu.VMEM((1, _LANES), jnp.float32)],
    )(x_flat, wr_pad)


def _offsets_body(cnt_ref, e0_ref, e1_ref, r0_ref, r1_ref, pos0_ref, pos1_ref,
                  bexp_ref):
    lane1 = lax.broadcasted_iota(jnp.int32, (1, _LANES), 1)
    cnt = cnt_ref[...].astype(jnp.int32)
    cnt = jnp.where(lane1 < _E, cnt, 0)
    nblk = (cnt + (_BM - 1)) // _BM
    tri = (lax.broadcasted_iota(jnp.int32, (_LANES, _LANES), 0)
           <= lax.broadcasted_iota(jnp.int32, (_LANES, _LANES), 1))
    endb = lax.dot_general(nblk.astype(jnp.float32), tri.astype(jnp.float32),
                           (((1,), (0,)), ((), ())),
                           preferred_element_type=jnp.float32)
    startb = endb - nblk.astype(jnp.float32)
    pstart = startb * jnp.float32(_BM)

    for e_ref, r_ref, p_ref in ((e0_ref, r0_ref, pos0_ref),
                                (e1_ref, r1_ref, pos1_ref)):
        e = e_ref[...]
        laneT = lax.broadcasted_iota(jnp.int32, (_T, _LANES), 1)
        base = jnp.sum(jnp.where(laneT == e, pstart, 0.0), axis=1,
                       keepdims=True)
        p_ref[...] = r_ref[...] + base.astype(jnp.int32)

    brow = lax.broadcasted_iota(jnp.int32, (_NBE, _LANES), 0)
    ge = (brow >= endb.astype(jnp.int32)) & (lane1 < _E)
    be = jnp.minimum(jnp.sum(ge.astype(jnp.int32), axis=1, keepdims=True),
                     _E - 1)
    used = jnp.sum(jnp.where(lane1 < _E, nblk, 0)).astype(jnp.int32)
    rowi = lax.broadcasted_iota(jnp.int32, (_NBE, 1), 0)
    bexp_ref[...] = jnp.where(rowi < _NB, be, used)


def _offsets_call(cnt, e0, e1, r0, r1):
    out_shapes = (
        jax.ShapeDtypeStruct((_T, 1), jnp.int32),
        jax.ShapeDtypeStruct((_T, 1), jnp.int32),
        jax.ShapeDtypeStruct((_NBE, 1), jnp.int32),
    )
    return pl.pallas_call(
        _offsets_body,
        out_shape=out_shapes,
    )(cnt, e0, e1, r0, r1)


def _dispatch_body(x_hbm, pos0_hbm, pos1_hbm, xg_hbm, idx0_v, idx1_v, rows_v,
                   sem):
    wid = lax.axis_index("s") * 2 + lax.axis_index("c")
    base = wid * _TW
    pltpu.sync_copy(pos0_hbm.at[wid], idx0_v)
    pltpu.sync_copy(pos1_hbm.at[wid], idx1_v)
    for j in range(_NCH):
        pltpu.sync_copy(x_hbm.at[pl.ds(base + j * _CD, _CD)], rows_v)
        cp0 = pltpu.make_async_copy(rows_v, xg_hbm.at[idx0_v.at[j]], sem)
        cp1 = pltpu.make_async_copy(rows_v, xg_hbm.at[idx1_v.at[j]], sem)
        cp0.start()
        cp1.start()
        cp0.wait()
        cp1.wait()


def _dispatch_call(x_flat, pos0w, pos1w):
    mesh = plsc.VectorSubcoreMesh(core_axis_name="c", subcore_axis_name="s")
    f = functools.partial(
        pl.kernel,
        out_type=jax.ShapeDtypeStruct((_NPAD, _D), jnp.float32),
        mesh=mesh,
        scratch_types=[
            pltpu.VMEM((_NCH, _CD), jnp.int32),
            pltpu.VMEM((_NCH, _CD), jnp.int32),
            pltpu.VMEM((_CD, _D), jnp.float32),
            pltpu.SemaphoreType.DMA,
        ],
    )(_dispatch_body)
    return f(x_flat, pos0w, pos1w)


def _ffn_body(bmap_ref, xg_ref, w1_ref, b1_ref, w2_ref, b2_ref, y_ref):
    b = pl.program_id(0)
    used = bmap_ref[_NB]

    @pl.when(b < used)
    def _():
        x = xg_ref[...].astype(jnp.bfloat16)
        acc = b2_ref[0]
        for ht in range(_H // _HT):
            w1c = w1_ref[0, ht * _HT:(ht + 1) * _HT, :]
            hpre = lax.dot_general(x, w1c, (((1,), (1,)), ((), ())),
                                   preferred_element_type=jnp.float32)
            hpre = hpre + b1_ref[0, :, ht * _HT:(ht + 1) * _HT]
            hact = _gelu(hpre).astype(jnp.bfloat16)
            w2c = w2_ref[0, :, ht * _HT:(ht + 1) * _HT]
            acc = acc + lax.dot_general(hact, w2c, (((1,), (1,)), ((), ())),
                                        preferred_element_type=jnp.float32)
        y_ref[...] = acc


def _ffn_call(bmap, xg, w1b, b1, w2b, b2):
    b1r = b1.reshape(_E, 1, _H)
    b2r = b2.reshape(_E, 1, _D)
    grid_spec = pltpu.PrefetchScalarGridSpec(
        num_scalar_prefetch=1,
        grid=(_NB,),
        in_specs=[
            pl.BlockSpec((_BM, _D), lambda b, m: (b, 0)),
            pl.BlockSpec((1, _H, _D), lambda b, m: (m[b], 0, 0)),
            pl.BlockSpec((1, 1, _H), lambda b, m: (m[b], 0, 0)),
            pl.BlockSpec((1, _D, _H), lambda b, m: (m[b], 0, 0)),
            pl.BlockSpec((1, 1, _D), lambda b, m: (m[b], 0, 0)),
        ],
        out_specs=pl.BlockSpec((_BM, _D), lambda b, m: (b, 0)),
    )
    return pl.pallas_call(
        _ffn_body,
        grid_spec=grid_spec,
        out_shape=jax.ShapeDtypeStruct((_NPAD, _D), jnp.float32),
    )(bmap, xg, w1b, b1r, w2b, b2r)


def _combine_body(y_hbm, pos0_hbm, pos1_hbm, y0_hbm, y1_hbm, idx0_v, idx1_v,
                  buf_v, sem):
    wid = lax.axis_index("s") * 2 + lax.axis_index("c")
    base = wid * _TW
    pltpu.sync_copy(pos0_hbm.at[wid], idx0_v)
    pltpu.sync_copy(pos1_hbm.at[wid], idx1_v)
    for j in range(_NCH):
        pltpu.make_async_copy(y_hbm.at[idx0_v.at[j]], buf_v, sem).start()
        pltpu.make_async_copy(y_hbm.at[idx0_v.at[j]], buf_v, sem).wait()
        pltpu.sync_copy(buf_v, y0_hbm.at[pl.ds(base + j * _CD, _CD)])
        pltpu.make_async_copy(y_hbm.at[idx1_v.at[j]], buf_v, sem).start()
        pltpu.make_async_copy(y_hbm.at[idx1_v.at[j]], buf_v, sem).wait()
        pltpu.sync_copy(buf_v, y1_hbm.at[pl.ds(base + j * _CD, _CD)])


def _combine_call(y, pos0w, pos1w):
    mesh = plsc.VectorSubcoreMesh(core_axis_name="c", subcore_axis_name="s")
    f = functools.partial(
        pl.kernel,
        out_type=(jax.ShapeDtypeStruct((_T, _D), jnp.float32),
                  jax.ShapeDtypeStruct((_T, _D), jnp.float32)),
        mesh=mesh,
        scratch_types=[
            pltpu.VMEM((_NCH, _CD), jnp.int32),
            pltpu.VMEM((_NCH, _CD), jnp.int32),
            pltpu.VMEM((_CD, _D), jnp.float32),
            pltpu.SemaphoreType.DMA,
        ],
    )(_combine_body)
    return f(y, pos0w, pos1w)


def _wsum_body(y0_ref, y1_ref, w0_ref, w1_ref, out_ref):
    out_ref[...] = y0_ref[...] * w0_ref[...] + y1_ref[...] * w1_ref[...]


def _wsum_call(y0, y1, w0, w1):
    nchunks = _T // _CH
    row_spec = pl.BlockSpec((_CH, _D), lambda c: (c, 0))
    col_spec = pl.BlockSpec((_CH, 1), lambda c: (c, 0))
    return pl.pallas_call(
        _wsum_body,
        grid=(nchunks,),
        in_specs=[row_spec, row_spec, col_spec, col_spec],
        out_specs=row_spec,
        out_shape=jax.ShapeDtypeStruct((_T, _D), jnp.float32),
    )(y0, y1, w0, w1)


def kernel(x, Wr, W1, b1, W2, b2):
    x_flat = x.reshape(_T, _D)
    wr_pad = jnp.zeros((_LANES, _D), jnp.float32).at[:_E].set(Wr)

    e0, e1, w0, w1, r0, r1, cnt = _router_call(x_flat, wr_pad)
    pos0, pos1, bexp = _offsets_call(cnt, e0, e1, r0, r1)

    bmap = bexp.reshape(_NBE)
    pos0w = pos0.reshape(_NW, _NCH, _CD)
    pos1w = pos1.reshape(_NW, _NCH, _CD)

    xg = _dispatch_call(x_flat, pos0w, pos1w)

    w1b = W1.astype(jnp.bfloat16)
    w2b = W2.astype(jnp.bfloat16)
    y = _ffn_call(bmap, xg, w1b, b1, w2b, b2)

    y0, y1 = _combine_call(y, pos0w, pos1w)
    out = _wsum_call(y0, y1, w0, w1)
    return out.reshape(_B, _S, _D)


# speedup vs baseline: 5.2387x; 1.0568x over previous
import functools

import jax
import jax.numpy as jnp
from jax import lax
from jax.experimental import pallas as pl
from jax.experimental.pallas import tpu as pltpu
from jax.experimental.pallas import tpu_sc as plsc

_B, _S, _D, _E, _K = 4, 2048, 1024, 8, 2
_T = _B * _S
_H = 4 * _D
_N = _T * _K
_LANES = 128

_BM = 512
_NB = _N // _BM + _E
_NBE = _NB + 8
_NPAD = _NB * _BM
_HT = 512

_CH = 1024

_WL = 128

_NW = 32
_TW = _T // _NW
_CD = 32
_NCH = _TW // _CD


def _erf(x):
    a1, a2, a3, a4, a5 = (0.254829592, -0.284496736, 1.421413741,
                          -1.453152027, 1.061405429)
    p = 0.3275911
    s = jnp.sign(x)
    ax = jnp.abs(x)
    t = 1.0 / (1.0 + p * ax)
    poly = t * (a1 + t * (a2 + t * (a3 + t * (a4 + t * a5))))
    y = 1.0 - poly * jnp.exp(-ax * ax)
    return s * y


def _gelu(x):
    u = 0.7978845608028654 * (x + 0.044715 * x * x * x)
    return 0.5 * x * (1.0 + jnp.tanh(u))


def _router_body(x_ref, wr_ref, e0_ref, e1_ref, wa_ref, wb_ref, r0_ref,
                 r1_ref, cnt_ref, carry):
    c = pl.program_id(0)

    @pl.when(c == 0)
    def _():
        carry[...] = jnp.zeros_like(carry)

    x = x_ref[...]
    logits = lax.dot_general(x, wr_ref[...], (((1,), (1,)), ((), ())),
                             preferred_element_type=jnp.float32)
    lane = lax.broadcasted_iota(jnp.int32, (_CH, _LANES), 1)
    neg = jnp.float32(-1e30)
    logits = jnp.where(lane < _E, logits, neg)

    m1 = jnp.max(logits, axis=1, keepdims=True)
    i1 = jnp.min(jnp.where(logits == m1, lane, _LANES), axis=1, keepdims=True)
    logits2 = jnp.where(lane == i1, neg, logits)
    m2 = jnp.max(logits2, axis=1, keepdims=True)
    i2 = jnp.min(jnp.where(logits2 == m2, lane, _LANES), axis=1, keepdims=True)

    s = jnp.exp(m2 - m1)
    w0 = 1.0 / (1.0 + s)
    w1 = 1.0 - w0

    ohA = (lane == i1)
    ohB = (lane == i2)
    ohAb = ohA.astype(jnp.bfloat16)
    ohBb = ohB.astype(jnp.bfloat16)
    row = lax.broadcasted_iota(jnp.int32, (_CH, _CH), 0)
    col = lax.broadcasted_iota(jnp.int32, (_CH, _CH), 1)
    lex = (col < row).astype(jnp.bfloat16)
    s0ex = lax.dot_general(lex, ohAb, (((1,), (0,)), ((), ())),
                           preferred_element_type=jnp.float32)
    s1ex = lax.dot_general(lex, ohBb, (((1,), (0,)), ((), ())),
                           preferred_element_type=jnp.float32)
    s0in = s0ex + ohA.astype(jnp.float32)

    carryv = carry[...]
    base0 = jnp.sum(jnp.where(ohA, carryv + s0ex + s1ex, 0.0), axis=1,
                    keepdims=True)
    base1 = jnp.sum(jnp.where(ohB, carryv + s0in + s1ex, 0.0), axis=1,
                    keepdims=True)

    e0_ref[...] = i1
    e1_ref[...] = i2
    wa_ref[...] = jnp.broadcast_to(w0, (_CH, _WL))
    wb_ref[...] = jnp.broadcast_to(w1, (_CH, _WL))
    r0_ref[...] = base0.astype(jnp.int32)
    r1_ref[...] = base1.astype(jnp.int32)

    newc = carryv + jnp.sum(ohA.astype(jnp.float32) + ohB.astype(jnp.float32),
                            axis=0, keepdims=True)
    carry[...] = newc
    cnt_ref[...] = newc


def _router_call(x_flat, wr_pad):
    nchunks = _T // _CH
    out_shapes = (
        jax.ShapeDtypeStruct((_T, 1), jnp.int32),
        jax.ShapeDtypeStruct((_T, 1), jnp.int32),
        jax.ShapeDtypeStruct((_T, _WL), jnp.float32),
        jax.ShapeDtypeStruct((_T, _WL), jnp.float32),
        jax.ShapeDtypeStruct((_T, 1), jnp.int32),
        jax.ShapeDtypeStruct((_T, 1), jnp.int32),
        jax.ShapeDtypeStruct((1, _LANES), jnp.float32),
    )
    col_spec = pl.BlockSpec((_CH, 1), lambda c: (c, 0))
    w_spec = pl.BlockSpec((_CH, _WL), lambda c: (c, 0))
    return pl.pallas_call(
        _router_body,
        grid=(nchunks,),
        in_specs=[
            pl.BlockSpec((_CH, _D), lambda c: (c, 0)),
            pl.BlockSpec((_LANES, _D), lambda c: (0, 0)),
        ],
        out_specs=(col_spec, col_spec, w_spec, w_spec, col_spec, col_spec,
                   pl.BlockSpec((1, _LANES), lambda c: (0, 0))),
        out_shape=out_shapes,
        scratch_shapes=[pltpu.VMEM((1, _LANES), jnp.float32)],
    )(x_flat, wr_pad)


def _offsets_body(cnt_ref, e0_ref, e1_ref, r0_ref, r1_ref, pos0_ref, pos1_ref,
                  bexp_ref):
    lane1 = lax.broadcasted_iota(jnp.int32, (1, _LANES), 1)
    cnt = cnt_ref[...].astype(jnp.int32)
    cnt = jnp.where(lane1 < _E, cnt, 0)
    nblk = (cnt + (_BM - 1)) // _BM
    tri = (lax.broadcasted_iota(jnp.int32, (_LANES, _LANES), 0)
           <= lax.broadcasted_iota(jnp.int32, (_LANES, _LANES), 1))
    endb = lax.dot_general(nblk.astype(jnp.float32), tri.astype(jnp.float32),
                           (((1,), (0,)), ((), ())),
                           preferred_element_type=jnp.float32)
    startb = endb - nblk.astype(jnp.float32)
    pstart = startb * jnp.float32(_BM)

    for e_ref, r_ref, p_ref in ((e0_ref, r0_ref, pos0_ref),
                                (e1_ref, r1_ref, pos1_ref)):
        e = e_ref[...]
        laneT = lax.broadcasted_iota(jnp.int32, (_T, _LANES), 1)
        base = jnp.sum(jnp.where(laneT == e, pstart, 0.0), axis=1,
                       keepdims=True)
        p_ref[...] = r_ref[...] + base.astype(jnp.int32)

    brow = lax.broadcasted_iota(jnp.int32, (_NBE, _LANES), 0)
    ge = (brow >= endb.astype(jnp.int32)) & (lane1 < _E)
    be = jnp.minimum(jnp.sum(ge.astype(jnp.int32), axis=1, keepdims=True),
                     _E - 1)
    used = jnp.sum(jnp.where(lane1 < _E, nblk, 0)).astype(jnp.int32)
    rowi = lax.broadcasted_iota(jnp.int32, (_NBE, 1), 0)
    bexp_ref[...] = jnp.where(rowi < _NB, be, used)


def _offsets_call(cnt, e0, e1, r0, r1):
    out_shapes = (
        jax.ShapeDtypeStruct((_T, 1), jnp.int32),
        jax.ShapeDtypeStruct((_T, 1), jnp.int32),
        jax.ShapeDtypeStruct((_NBE, 1), jnp.int32),
    )
    return pl.pallas_call(
        _offsets_body,
        out_shape=out_shapes,
    )(cnt, e0, e1, r0, r1)


def _dispatch_body(x_hbm, wr0_hbm, wr1_hbm, pos0_hbm, pos1_hbm, xg_hbm,
                   ws_hbm, idx0_v, idx1_v, rows_a, rows_b, wb0_a, wb0_b,
                   wb1_a, wb1_b, semr, semw):
    wid = lax.axis_index("s") * 2 + lax.axis_index("c")
    base = wid * _TW
    pltpu.sync_copy(pos0_hbm.at[wid], idx0_v)
    pltpu.sync_copy(pos1_hbm.at[wid], idx1_v)
    rows = (rows_a, rows_b)
    wb0 = (wb0_a, wb0_b)
    wb1 = (wb1_a, wb1_b)

    def start_reads(j, p):
        sl = pl.ds(base + j * _CD, _CD)
        return (pltpu.async_copy(x_hbm.at[sl], rows[p], semr),
                pltpu.async_copy(wr0_hbm.at[sl], wb0[p], semr),
                pltpu.async_copy(wr1_hbm.at[sl], wb1[p], semr))

    reads = start_reads(0, 0)
    writes = {}
    for j in range(_NCH):
        p = j % 2
        for cp in reads:
            cp.wait()
        if j + 1 < _NCH:
            if j >= 1:
                for cp in writes[j - 1]:
                    cp.wait()
            reads = start_reads(j + 1, p ^ 1)
        writes[j] = (
            pltpu.async_copy(rows[p], xg_hbm.at[idx0_v.at[j]], semw),
            pltpu.async_copy(rows[p], xg_hbm.at[idx1_v.at[j]], semw),
            pltpu.async_copy(wb0[p], ws_hbm.at[idx0_v.at[j]], semw),
            pltpu.async_copy(wb1[p], ws_hbm.at[idx1_v.at[j]], semw),
        )
    for j in (_NCH - 2, _NCH - 1):
        for cp in writes[j]:
            cp.wait()


def _dispatch_call(x_flat, wrow0, wrow1, pos0w, pos1w):
    mesh = plsc.VectorSubcoreMesh(core_axis_name="c", subcore_axis_name="s")
    f = functools.partial(
        pl.kernel,
        out_type=(jax.ShapeDtypeStruct((_NPAD, _D), jnp.float32),
                  jax.ShapeDtypeStruct((_NPAD, _WL), jnp.float32)),
        mesh=mesh,
        scratch_types=[
            pltpu.VMEM((_NCH, _CD), jnp.int32),
            pltpu.VMEM((_NCH, _CD), jnp.int32),
            pltpu.VMEM((_CD, _D), jnp.float32),
            pltpu.VMEM((_CD, _D), jnp.float32),
            pltpu.VMEM((_CD, _WL), jnp.float32),
            pltpu.VMEM((_CD, _WL), jnp.float32),
            pltpu.VMEM((_CD, _WL), jnp.float32),
            pltpu.VMEM((_CD, _WL), jnp.float32),
            pltpu.SemaphoreType.DMA,
            pltpu.SemaphoreType.DMA,
        ],
    )(_dispatch_body)
    return f(x_flat, wrow0, wrow1, pos0w, pos1w)


def _ffn_body(bmap_ref, xg_ref, ws_ref, w1_ref, b1_ref, w2_ref, b2_ref,
              y_ref):
    b = pl.program_id(0)
    used = bmap_ref[_NB]

    @pl.when(b < used)
    def _():
        x = xg_ref[...].astype(jnp.bfloat16)
        acc = b2_ref[0]
        for ht in range(_H // _HT):
            w1c = w1_ref[0, ht * _HT:(ht + 1) * _HT, :]
            hpre = lax.dot_general(x, w1c, (((1,), (1,)), ((), ())),
                                   preferred_element_type=jnp.float32)
            hpre = hpre + b1_ref[0, :, ht * _HT:(ht + 1) * _HT]
            hact = _gelu(hpre).astype(jnp.bfloat16)
            w2c = w2_ref[0, :, ht * _HT:(ht + 1) * _HT]
            acc = acc + lax.dot_general(hact, w2c, (((1,), (1,)), ((), ())),
                                        preferred_element_type=jnp.float32)
        y_ref[...] = acc * ws_ref[:, 0:1]


def _ffn_call(bmap, xg, wsort, w1b, b1, w2b, b2):
    b1r = b1.reshape(_E, 1, _H)
    b2r = b2.reshape(_E, 1, _D)
    grid_spec = pltpu.PrefetchScalarGridSpec(
        num_scalar_prefetch=1,
        grid=(_NB,),
        in_specs=[
            pl.BlockSpec((_BM, _D), lambda b, m: (b, 0)),
            pl.BlockSpec((_BM, _WL), lambda b, m: (b, 0)),
            pl.BlockSpec((1, _H, _D), lambda b, m: (m[b], 0, 0)),
            pl.BlockSpec((1, 1, _H), lambda b, m: (m[b], 0, 0)),
            pl.BlockSpec((1, _D, _H), lambda b, m: (m[b], 0, 0)),
            pl.BlockSpec((1, 1, _D), lambda b, m: (m[b], 0, 0)),
        ],
        out_specs=pl.BlockSpec((_BM, _D), lambda b, m: (b, 0)),
    )
    return pl.pallas_call(
        _ffn_body,
        grid_spec=grid_spec,
        out_shape=jax.ShapeDtypeStruct((_NPAD, _D), jnp.float32),
    )(bmap, xg, wsort, w1b, b1r, w2b, b2r)


def _combine_body(y_hbm, pos0_hbm, pos1_hbm, out_hbm, idx0_v, idx1_v, buf_a,
                  buf_b, sema, semb):
    wid = lax.axis_index("s") * 2 + lax.axis_index("c")
    base = wid * _TW
    pltpu.sync_copy(pos0_hbm.at[wid], idx0_v)
    pltpu.sync_copy(pos1_hbm.at[wid], idx1_v)
    bufs = (buf_a, buf_b)
    g0 = pltpu.async_copy(y_hbm.at[idx0_v.at[0]], bufs[0], sema)
    for j in range(_NCH):
        p = j % 2
        g0.wait()
        gadd = pltpu.async_copy(y_hbm.at[idx1_v.at[j]], bufs[p], semb,
                                add=True)
        if j + 1 < _NCH:
            g0 = pltpu.async_copy(y_hbm.at[idx0_v.at[j + 1]], bufs[p ^ 1],
                                  sema)
        gadd.wait()
        pltpu.sync_copy(bufs[p], out_hbm.at[pl.ds(base + j * _CD, _CD)])


def _combine_call(y, pos0w, pos1w):
    mesh = plsc.VectorSubcoreMesh(core_axis_name="c", subcore_axis_name="s")
    f = functools.partial(
        pl.kernel,
        out_type=jax.ShapeDtypeStruct((_T, _D), jnp.float32),
        mesh=mesh,
        scratch_types=[
            pltpu.VMEM((_NCH, _CD), jnp.int32),
            pltpu.VMEM((_NCH, _CD), jnp.int32),
            pltpu.VMEM((_CD, _D), jnp.float32),
            pltpu.VMEM((_CD, _D), jnp.float32),
            pltpu.SemaphoreType.DMA,
            pltpu.SemaphoreType.DMA,
        ],
    )(_combine_body)
    return f(y, pos0w, pos1w)


def kernel(x, Wr, W1, b1, W2, b2):
    x_flat = x.reshape(_T, _D)
    wr_pad = jnp.zeros((_LANES, _D), jnp.float32).at[:_E].set(Wr)

    e0, e1, w0, w1, r0, r1, cnt = _router_call(x_flat, wr_pad)
    pos0, pos1, bexp = _offsets_call(cnt, e0, e1, r0, r1)

    bmap = bexp.reshape(_NBE)
    pos0w = pos0.reshape(_NW, _NCH, _CD)
    pos1w = pos1.reshape(_NW, _NCH, _CD)

    xg, wsort = _dispatch_call(x_flat, w0, w1, pos0w, pos1w)

    w1b = W1.astype(jnp.bfloat16)
    w2b = W2.astype(jnp.bfloat16)
    y = _ffn_call(bmap, xg, wsort, w1b, b1, w2b, b2)

    out = _combine_call(y, pos0w, pos1w)
    return out.reshape(_B, _S, _D)
